# baseline (speedup 1.0000x reference)
import functools

import jax
import jax.numpy as jnp
from jax import lax
from jax.experimental import pallas as pl
from jax.experimental.pallas import tpu as pltpu
from jax.experimental.pallas import tpu_sc as plsc

B = 16384
D = 32
NC, NS, L = 2, 16, 16
NW = NC * NS
BPW = B // NW
OR = B // 4


def _sc_gather(user_embedding, item_embedding, u, i, j):
    mesh = plsc.VectorSubcoreMesh(core_axis_name="c", subcore_axis_name="s")

    @functools.partial(
        pl.kernel,
        mesh=mesh,
        out_type=[
            jax.ShapeDtypeStruct((B, D), jnp.float32),
            jax.ShapeDtypeStruct((B, D), jnp.float32),
            jax.ShapeDtypeStruct((B, D), jnp.float32),
        ],
        scratch_types=[
            pltpu.VMEM((BPW,), jnp.int32),
            pltpu.VMEM((BPW, D), jnp.float32),
            pltpu.SemaphoreType.DMA,
            pltpu.SemaphoreType.DMA,
            pltpu.SemaphoreType.DMA,
            pltpu.SemaphoreType.DMA,
        ],
    )
    def k(tu_hbm, ti_hbm, u_hbm, i_hbm, j_hbm, gu_hbm, gp_hbm, gn_hbm,
          idx_s, stage, sem, sem1, sem2, sem3):
        wid = lax.axis_index("s") * NC + lax.axis_index("c")
        base = wid * BPW

        def gather(idx_hbm, table, out_hbm):
            pltpu.sync_copy(idx_hbm.at[pl.ds(base, BPW)], idx_s)

            sems = (sem, sem1, sem2, sem3)

            @plsc.parallel_loop(0, BPW, step=L)
            def body(rb):
                v = idx_s[pl.ds(rb, L)]
                for r in range(L):
                    pltpu.async_copy(
                        table.at[pl.ds(v[r], 1), :],
                        stage.at[pl.ds(rb + r, 1), :],
                        sems[r % 4])
            for q in range(4):
                pltpu.make_async_copy(
                    table.at[pl.ds(0, BPW // 4), :],
                    stage.at[pl.ds(q * (BPW // 4), BPW // 4), :],
                    sems[q]).wait()
            pltpu.sync_copy(stage, out_hbm.at[pl.ds(base, BPW), :])

        gather(u_hbm, tu_hbm, gu_hbm)
        gather(i_hbm, ti_hbm, gp_hbm)
        gather(j_hbm, ti_hbm, gn_hbm)

    return k(user_embedding, item_embedding, u, i, j)


def _tc_reduce(gu, gp, gn):
    def body(u_ref, p_ref, n_ref, bpr_ref, reg_ref):
        un = u_ref[...]
        pn = p_ref[...]
        nn = n_ref[...]
        h = jnp.sum(un * (nn - pn), axis=1)
        sp = jnp.maximum(h, 0.0) + jnp.log(1.0 + jnp.exp(-jnp.abs(h)))
        bpr = jnp.sum(sp)
        reg = jnp.sum(un * un + pn * pn + nn * nn) * (1.0 / B)
        bpr_ref[...] = jnp.full((8, 128), bpr, jnp.float32)
        reg_ref[...] = jnp.full((8, 128), reg, jnp.float32)

    bpr, reg = pl.pallas_call(
        body,
        out_shape=[jax.ShapeDtypeStruct((8, 128), jnp.float32),
                   jax.ShapeDtypeStruct((8, 128), jnp.float32)],
    )(gu, gp, gn)
    return bpr[0, 0], reg[0, 0]


def kernel(user_embedding, item_embedding, u, i, j):
    u = u.astype(jnp.int32)
    i = i.astype(jnp.int32)
    j = j.astype(jnp.int32)
    gu, gp, gn = _sc_gather(user_embedding, item_embedding, u, i, j)
    return _tc_reduce(gu, gp, gn)
